# baseline (speedup 1.0000x reference)
import functools

import numpy as np
import jax
import jax.numpy as jnp
from jax import lax
from jax.experimental import pallas as pl
from jax.experimental.pallas import tpu as pltpu
from jax.experimental.pallas import tpu_sc as plsc

_B = 64
_V = 100000
_SUB = 8
_W = 1024
_NCH = _V // _W
_TAIL = _V - _NCH * _W

_TC_ROWS = 48
_SC_ROW0 = 48
_SC_ROWS = _B - _SC_ROW0
_HALF = _V // 2
_SC_UNROLL = 25
_SC_NV = _HALF // (16 * _SC_UNROLL)

_KS = (np.uint32(0), np.uint32(42), np.uint32(0x1BD11BDA ^ 42))
_ROTS = ((13, 15, 26, 6), (17, 29, 16, 24))
_TINY = np.float32(np.finfo(np.float32).tiny)
_BIG_IDX = np.int32(0x7FFFFFFF)

_LOG2_COEF = tuple(
    np.float32(c)
    for c in (
        1.442695, -0.7213481, 0.48089954, -0.36063376, 0.28845695,
        -0.24115817, 0.2078914, -0.17613588, 0.14491563, -0.1456088,
        0.18077716, -0.17387132, 0.0942453, -0.021125063,
    )
)


def _rotl(x, d):
    return lax.shift_left(x, np.uint32(d)) | lax.shift_right_logical(
        x, np.uint32(32 - d)
    )


def _threefry_xor(cnt):
    x1 = cnt + _KS[1]
    x0 = x1
    x1 = x0 ^ _rotl(x1, 13)
    for r in (15, 26, 6):
        x0 = x0 + x1
        x1 = x0 ^ _rotl(x1, r)
    x0 = x0 + _KS[1]
    x1 = x1 + np.uint32(_KS[2] + np.uint32(1))
    for blk in range(1, 5):
        for r in _ROTS[blk % 2]:
            x0 = x0 + x1
            x1 = x0 ^ _rotl(x1, r)
        x0 = x0 + _KS[(blk + 1) % 3]
        x1 = x1 + np.uint32(_KS[(blk + 2) % 3] + np.uint32(blk + 1))
    return x0 ^ x1


def _uniform_from_cnt(cnt_i32):
    bits = _threefry_xor(cnt_i32.astype(jnp.uint32))
    return (
        lax.bitcast_convert_type(
            lax.shift_right_logical(bits, np.uint32(9)) | np.uint32(0x3F800000),
            jnp.float32,
        )
        - np.float32(1.0)
    )


def _race_d(xs, cnt_i32, ntinv2):
    u = jnp.maximum(_uniform_from_cnt(cnt_i32), _TINY)
    return jnp.log2(u) * jnp.exp2(xs * ntinv2)


def _rowgroup_body(x_ref, tinv_ref, o_ref):
    rg = pl.program_id(0)
    ntinv2 = tinv_ref[...]
    row0 = rg * np.int32(_SUB)
    s_iota = lax.broadcasted_iota(jnp.int32, (_SUB, _W), 0)
    l_iota = lax.broadcasted_iota(jnp.int32, (_SUB, _W), 1)
    base = (row0 + s_iota) * np.int32(_V)
    cnt0 = base + l_iota

    def body(j, carry):
        vmax, vcnt = carry
        for h in range(4):
            col = j * np.int32(4 * _W) + np.int32(h * _W)
            xs = x_ref[:, pl.ds(col, _W)]
            cnt = cnt0 + col
            d = _race_d(xs, cnt, ntinv2)
            upd = d > vmax
            vmax = jnp.where(upd, d, vmax)
            vcnt = jnp.where(upd, cnt, vcnt)
        return (vmax, vcnt)

    vmax0 = jnp.full((_SUB, _W), np.float32(-np.inf), jnp.float32)
    vmax, vcnt = lax.fori_loop(0, _NCH // 4, body, (vmax0, cnt0))

    col96 = np.int32((_NCH - 1) * _W)
    xs96 = x_ref[:, (_NCH - 1) * _W : _NCH * _W]
    cnt96 = cnt0 + col96
    d96 = _race_d(xs96, cnt96, ntinv2)
    upd = d96 > vmax
    vmax = jnp.where(upd, d96, vmax)
    vcnt = jnp.where(upd, cnt96, vcnt)

    m1 = jnp.max(vmax, axis=1, keepdims=True)
    sel1 = jnp.min(
        jnp.where(vmax == m1, vcnt, _BIG_IDX), axis=1, keepdims=True
    )

    xs_t = x_ref[:, _NCH * _W : _V]
    s_t = lax.broadcasted_iota(jnp.int32, (_SUB, _TAIL), 0)
    l_t = lax.broadcasted_iota(jnp.int32, (_SUB, _TAIL), 1)
    cnt_t = (row0 + s_t) * np.int32(_V) + l_t + np.int32(_NCH * _W)
    d_t = _race_d(xs_t, cnt_t, ntinv2)
    m2 = jnp.max(d_t, axis=1, keepdims=True)
    sel2 = jnp.min(
        jnp.where(d_t == m2, cnt_t, _BIG_IDX), axis=1, keepdims=True
    )

    sel = jnp.where(
        m2 > m1, sel2, jnp.where(m1 > m2, sel1, jnp.minimum(sel1, sel2))
    )
    col = sel - (row0 + lax.broadcasted_iota(jnp.int32, (_SUB, 1), 0)) * np.int32(_V)
    o_ref[...] = jnp.broadcast_to(col, (_SUB, 128))


def _log2_sc(u):
    ub = lax.bitcast_convert_type(u, jnp.int32)
    e_int = lax.shift_right_logical(ub, np.int32(23)) - np.int32(127)
    m = lax.bitcast_convert_type(
        (ub & np.int32(0x007FFFFF)) | np.int32(0x3F800000), jnp.float32
    )
    big = u >= np.float32(0.5)
    t = jnp.where(big, u, m) - np.float32(1.0)
    base = jnp.where(big, np.float32(0.0), e_int.astype(jnp.float32))
    acc = jnp.full_like(t, _LOG2_COEF[-1])
    for c in _LOG2_COEF[-2::-1]:
        acc = acc * t + c
    return base + t * acc


def _sc_body(x_hbm, nt_hbm, score_out, col_out, x_v, nt_v, s_v, i_v):
    c = lax.axis_index("c")
    s = lax.axis_index("s")
    wid = s * np.int32(2) + c
    rowl = lax.shift_right_logical(wid, 1)
    half = wid & np.int32(1)
    row = rowl + np.int32(_SC_ROW0)
    pltpu.sync_copy(x_hbm.at[pl.ds(wid * np.int32(_HALF), _HALF)], x_v)
    pltpu.sync_copy(nt_hbm.at[pl.ds(wid * np.int32(16), 16)], nt_v)
    nt = nt_v[...]
    cnt_base = row * np.int32(_V) + half * np.int32(_HALF) + lax.iota(jnp.int32, 16)

    def body(j, carry):
        vmax, vcnt = carry
        for hh in range(_SC_UNROLL):
            jj = j * np.int32(_SC_UNROLL) + np.int32(hh)
            off = jj * np.int32(16)
            xs = x_v[pl.ds(off, 16)]
            cnt = cnt_base + off
            u = _uniform_from_cnt(cnt)
            d = _log2_sc(u) * jnp.exp(xs * nt)
            upd = d > vmax
            vmax = jnp.where(upd, d, vmax)
            vcnt = jnp.where(upd, cnt, vcnt)
        return (vmax, vcnt)

    vmax0 = jnp.full((16,), np.float32(-np.inf), jnp.float32)
    vcnt0 = jnp.zeros((16,), jnp.int32)
    vmax, vcnt = lax.fori_loop(0, _SC_NV, body, (vmax0, vcnt0))

    s_v[...] = vmax
    i_v[...] = vcnt
    pltpu.sync_copy(s_v, score_out.at[pl.ds(wid * np.int32(16), 16)])
    pltpu.sync_copy(i_v, col_out.at[pl.ds(wid * np.int32(16), 16)])


_sc_kernel = functools.partial(
    pl.kernel,
    out_type=(
        jax.ShapeDtypeStruct((2 * _SC_ROWS * 16,), jnp.float32),
        jax.ShapeDtypeStruct((2 * _SC_ROWS * 16,), jnp.int32),
    ),
    mesh=plsc.VectorSubcoreMesh(core_axis_name="c", subcore_axis_name="s"),
    scratch_types=[
        pltpu.VMEM((_HALF,), jnp.float32),
        pltpu.VMEM((16,), jnp.float32),
        pltpu.VMEM((16,), jnp.float32),
        pltpu.VMEM((16,), jnp.int32),
    ],
)(_sc_body)


def kernel(logits, temperatures):
    ntinv = np.float32(-1.0) / temperatures
    tinv2 = (np.float32(np.log2(np.e)) * ntinv).reshape(_B, 1)

    nt_sc = jnp.broadcast_to(
        jnp.repeat(ntinv[_SC_ROW0:], 2)[:, None], (2 * _SC_ROWS, 16)
    ).reshape(-1)
    x_sc = logits[_SC_ROW0:].reshape(-1)
    sc_score, sc_cnt = _sc_kernel(x_sc, nt_sc)

    tc_out = pl.pallas_call(
        _rowgroup_body,
        grid=(_TC_ROWS // _SUB,),
        in_specs=[
            pl.BlockSpec((_SUB, _V), lambda g: (g, 0)),
            pl.BlockSpec((_SUB, 1), lambda g: (g, 0)),
        ],
        out_specs=pl.BlockSpec((_SUB, 128), lambda g: (g, 0)),
        out_shape=jax.ShapeDtypeStruct((_TC_ROWS, 128), jnp.int32),
    )(logits, tinv2)
    sc_s = sc_score.reshape(_SC_ROWS, 32)
    sc_c = sc_cnt.reshape(_SC_ROWS, 32)
    m = jnp.max(sc_s, axis=1, keepdims=True)
    sel = jnp.min(jnp.where(sc_s == m, sc_c, _BIG_IDX), axis=1)
    rows = np.arange(_SC_ROW0, _B, dtype=np.int32)
    sc_idx = sel - rows * np.int32(_V)
    return jnp.concatenate([tc_out[:, 0], sc_idx])


# speedup vs baseline: 1.2413x; 1.2413x over previous
import numpy as np
import jax
import jax.numpy as jnp
from jax import lax
from jax.experimental import pallas as pl

_B = 64
_V = 100000
_SUB = 8
_W = 1024
_NCH = _V // _W
_TAIL = _V - _NCH * _W

_KS = (np.uint32(0), np.uint32(42), np.uint32(0x1BD11BDA ^ 42))
_ROTS = ((13, 15, 26, 6), (17, 29, 16, 24))
_TINY = np.float32(np.finfo(np.float32).tiny)
_BIG_IDX = np.int32(0x7FFFFFFF)


def _rotl(x, d):
    return lax.shift_left(x, np.uint32(d)) | lax.shift_right_logical(
        x, np.uint32(32 - d)
    )


def _threefry_xor(cnt):
    x1 = cnt + _KS[1]
    x0 = x1
    x1 = x0 ^ _rotl(x1, 13)
    for r in (15, 26, 6):
        x0 = x0 + x1
        x1 = x0 ^ _rotl(x1, r)
    x0 = x0 + _KS[1]
    x1 = x1 + np.uint32(_KS[2] + np.uint32(1))
    for blk in range(1, 5):
        for r in _ROTS[blk % 2]:
            x0 = x0 + x1
            x1 = x0 ^ _rotl(x1, r)
        x0 = x0 + _KS[(blk + 1) % 3]
        x1 = x1 + np.uint32(_KS[(blk + 2) % 3] + np.uint32(blk + 1))
    return x0 ^ x1


def _race_d(xs, cnt_i32, ntinv2):
    bits = _threefry_xor(cnt_i32.astype(jnp.uint32))
    fl = (
        lax.bitcast_convert_type(
            lax.shift_right_logical(bits, np.uint32(9)) | np.uint32(0x3F800000),
            jnp.float32,
        )
        - np.float32(1.0)
    )
    u = jnp.maximum(fl, _TINY)
    return jnp.log2(u) * jnp.exp2(xs * ntinv2)


def _rowgroup_body(x_ref, tinv_ref, o_ref):
    rg = pl.program_id(0)
    ntinv2 = tinv_ref[...]
    row0 = rg * np.int32(_SUB)
    s_iota = lax.broadcasted_iota(jnp.int32, (_SUB, _W), 0)
    l_iota = lax.broadcasted_iota(jnp.int32, (_SUB, _W), 1)
    base = (row0 + s_iota) * np.int32(_V)
    cnt0 = base + l_iota

    def body(j, carry):
        vmax, vcnt = carry
        for h in range(4):
            col = j * np.int32(4 * _W) + np.int32(h * _W)
            xs = x_ref[:, pl.ds(col, _W)]
            cnt = cnt0 + col
            d = _race_d(xs, cnt, ntinv2)
            upd = d > vmax
            vmax = jnp.where(upd, d, vmax)
            vcnt = jnp.where(upd, cnt, vcnt)
        return (vmax, vcnt)

    vmax0 = jnp.full((_SUB, _W), np.float32(-np.inf), jnp.float32)
    vmax, vcnt = lax.fori_loop(0, _NCH // 4, body, (vmax0, cnt0))

    col96 = np.int32((_NCH - 1) * _W)
    xs96 = x_ref[:, (_NCH - 1) * _W : _NCH * _W]
    cnt96 = cnt0 + col96
    d96 = _race_d(xs96, cnt96, ntinv2)
    upd = d96 > vmax
    vmax = jnp.where(upd, d96, vmax)
    vcnt = jnp.where(upd, cnt96, vcnt)

    m1 = jnp.max(vmax, axis=1, keepdims=True)
    sel1 = jnp.min(
        jnp.where(vmax == m1, vcnt, _BIG_IDX), axis=1, keepdims=True
    )

    xs_t = x_ref[:, _NCH * _W : _V]
    s_t = lax.broadcasted_iota(jnp.int32, (_SUB, _TAIL), 0)
    l_t = lax.broadcasted_iota(jnp.int32, (_SUB, _TAIL), 1)
    cnt_t = (row0 + s_t) * np.int32(_V) + l_t + np.int32(_NCH * _W)
    d_t = _race_d(xs_t, cnt_t, ntinv2)
    m2 = jnp.max(d_t, axis=1, keepdims=True)
    sel2 = jnp.min(
        jnp.where(d_t == m2, cnt_t, _BIG_IDX), axis=1, keepdims=True
    )

    sel = jnp.where(
        m2 > m1, sel2, jnp.where(m1 > m2, sel1, jnp.minimum(sel1, sel2))
    )
    col = sel - (row0 + lax.broadcasted_iota(jnp.int32, (_SUB, 1), 0)) * np.int32(_V)
    o_ref[...] = jnp.broadcast_to(col, (_SUB, 128))


def kernel(logits, temperatures):
    tinv = (np.float32(-np.log2(np.e)) / temperatures).reshape(_B, 1)
    out = pl.pallas_call(
        _rowgroup_body,
        grid=(_B // _SUB,),
        in_specs=[
            pl.BlockSpec((_SUB, _V), lambda g: (g, 0)),
            pl.BlockSpec((_SUB, 1), lambda g: (g, 0)),
        ],
        out_specs=pl.BlockSpec((_SUB, 128), lambda g: (g, 0)),
        out_shape=jax.ShapeDtypeStruct((_B, 128), jnp.int32),
    )(logits, tinv)
    return out[:, 0]


# speedup vs baseline: 1.2590x; 1.0142x over previous
import numpy as np
import jax
import jax.numpy as jnp
from jax import lax
from jax.experimental import pallas as pl

_B = 64
_V = 100000
_SUB = 8
_W = 1024
_NCH = _V // _W
_TAIL = _V - _NCH * _W

_KS = (np.uint32(0), np.uint32(42), np.uint32(0x1BD11BDA ^ 42))
_ROTS = ((13, 15, 26, 6), (17, 29, 16, 24))
_TINY = np.float32(np.finfo(np.float32).tiny)
_BIG_IDX = np.int32(0x7FFFFFFF)


def _rotl(x, d):
    return lax.shift_left(x, np.uint32(d)) | lax.shift_right_logical(
        x, np.uint32(32 - d)
    )


def _threefry_xor(cnt):
    x1 = cnt + _KS[1]
    x0 = x1
    x1 = x0 ^ _rotl(x1, 13)
    for r in (15, 26, 6):
        x0 = x0 + x1
        x1 = x0 ^ _rotl(x1, r)
    x0 = x0 + _KS[1]
    x1 = x1 + np.uint32(_KS[2] + np.uint32(1))
    for blk in range(1, 5):
        for r in _ROTS[blk % 2]:
            x0 = x0 + x1
            x1 = x0 ^ _rotl(x1, r)
        x0 = x0 + _KS[(blk + 1) % 3]
        x1 = x1 + np.uint32(_KS[(blk + 2) % 3] + np.uint32(blk + 1))
    return x0 ^ x1


def _race_d(xs, cnt_i32, ntinv2):
    bits = _threefry_xor(cnt_i32.astype(jnp.uint32))
    fl = (
        lax.bitcast_convert_type(
            lax.shift_right_logical(bits, np.uint32(9)) | np.uint32(0x3F800000),
            jnp.float32,
        )
        - np.float32(1.0)
    )
    return jnp.log2(fl) * jnp.exp2(xs * ntinv2)


def _rowgroup_body(x_ref, tinv_ref, o_ref):
    rg = pl.program_id(0)
    ntinv2 = tinv_ref[...]
    row0 = rg * np.int32(_SUB)
    s_iota = lax.broadcasted_iota(jnp.int32, (_SUB, _W), 0)
    l_iota = lax.broadcasted_iota(jnp.int32, (_SUB, _W), 1)
    base = (row0 + s_iota) * np.int32(_V)
    cnt0 = base + l_iota

    def body(j, carry):
        vmax, vcnt = carry
        for h in range(6):
            col = j * np.int32(6 * _W) + np.int32(h * _W)
            xs = x_ref[:, pl.ds(col, _W)]
            cnt = cnt0 + col
            d = _race_d(xs, cnt, ntinv2)
            upd = d > vmax
            vmax = jnp.where(upd, d, vmax)
            vcnt = jnp.where(upd, cnt, vcnt)
        return (vmax, vcnt)

    vmax0 = jnp.full((_SUB, _W), np.float32(-np.inf), jnp.float32)
    vmax, vcnt = lax.fori_loop(0, (_NCH - 1) // 6, body, (vmax0, cnt0))

    col96 = np.int32((_NCH - 1) * _W)
    xs96 = x_ref[:, (_NCH - 1) * _W : _NCH * _W]
    cnt96 = cnt0 + col96
    d96 = _race_d(xs96, cnt96, ntinv2)
    upd = d96 > vmax
    vmax = jnp.where(upd, d96, vmax)
    vcnt = jnp.where(upd, cnt96, vcnt)

    m1 = jnp.max(vmax, axis=1, keepdims=True)
    sel1 = jnp.min(
        jnp.where(vmax == m1, vcnt, _BIG_IDX), axis=1, keepdims=True
    )

    xs_t = x_ref[:, _NCH * _W : _V]
    s_t = lax.broadcasted_iota(jnp.int32, (_SUB, _TAIL), 0)
    l_t = lax.broadcasted_iota(jnp.int32, (_SUB, _TAIL), 1)
    cnt_t = (row0 + s_t) * np.int32(_V) + l_t + np.int32(_NCH * _W)
    d_t = _race_d(xs_t, cnt_t, ntinv2)
    m2 = jnp.max(d_t, axis=1, keepdims=True)
    sel2 = jnp.min(
        jnp.where(d_t == m2, cnt_t, _BIG_IDX), axis=1, keepdims=True
    )

    sel = jnp.where(
        m2 > m1, sel2, jnp.where(m1 > m2, sel1, jnp.minimum(sel1, sel2))
    )
    col = sel - (row0 + lax.broadcasted_iota(jnp.int32, (_SUB, 1), 0)) * np.int32(_V)
    o_ref[...] = jnp.broadcast_to(col, (_SUB, 128))


def kernel(logits, temperatures):
    tinv = (np.float32(-np.log2(np.e)) / temperatures).reshape(_B, 1)
    out = pl.pallas_call(
        _rowgroup_body,
        grid=(_B // _SUB,),
        in_specs=[
            pl.BlockSpec((_SUB, _V), lambda g: (g, 0)),
            pl.BlockSpec((_SUB, 1), lambda g: (g, 0)),
        ],
        out_specs=pl.BlockSpec((_SUB, 128), lambda g: (g, 0)),
        out_shape=jax.ShapeDtypeStruct((_B, 128), jnp.int32),
    )(logits, tinv)
    return out[:, 0]


# speedup vs baseline: 1.2608x; 1.0014x over previous
import numpy as np
import jax
import jax.numpy as jnp
from jax import lax
from jax.experimental import pallas as pl

_B = 64
_V = 100000
_SUB = 8
_W = 1024
_NCH = _V // _W
_TAIL = _V - _NCH * _W

_KS = (np.uint32(0), np.uint32(42), np.uint32(0x1BD11BDA ^ 42))
_ROTS = ((13, 15, 26, 6), (17, 29, 16, 24))
_TINY = np.float32(np.finfo(np.float32).tiny)
_BIG_IDX = np.int32(0x7FFFFFFF)


def _rotl(x, d):
    return lax.shift_left(x, np.uint32(d)) | lax.shift_right_logical(
        x, np.uint32(32 - d)
    )


def _threefry_xor(cnt):
    x1 = cnt + _KS[1]
    x0 = x1
    x1 = x0 ^ _rotl(x1, 13)
    for r in (15, 26, 6):
        x0 = x0 + x1
        x1 = x0 ^ _rotl(x1, r)
    x0 = x0 + _KS[1]
    x1 = x1 + np.uint32(_KS[2] + np.uint32(1))
    for blk in range(1, 5):
        for r in _ROTS[blk % 2]:
            x0 = x0 + x1
            x1 = x0 ^ _rotl(x1, r)
        x0 = x0 + _KS[(blk + 1) % 3]
        x1 = x1 + np.uint32(_KS[(blk + 2) % 3] + np.uint32(blk + 1))
    return x0 ^ x1


def _race_d(xs, cnt_i32, ntinv2):
    bits = _threefry_xor(cnt_i32.astype(jnp.uint32))
    fl = (
        lax.bitcast_convert_type(
            lax.shift_right_logical(bits, np.uint32(9)) | np.uint32(0x3F800000),
            jnp.float32,
        )
        - np.float32(1.0)
    )
    return jnp.log2(fl) * jnp.exp2(xs * ntinv2)


def _rowgroup_body(x_ref, tinv_ref, o_ref):
    rg = pl.program_id(0)
    ntinv2 = tinv_ref[...]
    row0 = rg * np.int32(_SUB)
    s_iota = lax.broadcasted_iota(jnp.int32, (_SUB, _W), 0)
    l_iota = lax.broadcasted_iota(jnp.int32, (_SUB, _W), 1)
    base = (row0 + s_iota) * np.int32(_V)
    cnt0 = base + l_iota

    def body(j, carry):
        vmax, vcnt = carry
        for h in range(8):
            col = j * np.int32(8 * _W) + np.int32(h * _W)
            xs = x_ref[:, pl.ds(col, _W)]
            cnt = cnt0 + col
            d = _race_d(xs, cnt, ntinv2)
            upd = d > vmax
            vmax = jnp.where(upd, d, vmax)
            vcnt = jnp.where(upd, cnt, vcnt)
        return (vmax, vcnt)

    vmax0 = jnp.full((_SUB, _W), np.float32(-np.inf), jnp.float32)
    vmax, vcnt = lax.fori_loop(0, (_NCH - 1) // 8, body, (vmax0, cnt0))

    col96 = np.int32((_NCH - 1) * _W)
    xs96 = x_ref[:, (_NCH - 1) * _W : _NCH * _W]
    cnt96 = cnt0 + col96
    d96 = _race_d(xs96, cnt96, ntinv2)
    upd = d96 > vmax
    vmax = jnp.where(upd, d96, vmax)
    vcnt = jnp.where(upd, cnt96, vcnt)

    m1 = jnp.max(vmax, axis=1, keepdims=True)
    sel1 = jnp.min(
        jnp.where(vmax == m1, vcnt, _BIG_IDX), axis=1, keepdims=True
    )

    xs_t = x_ref[:, _NCH * _W : _V]
    s_t = lax.broadcasted_iota(jnp.int32, (_SUB, _TAIL), 0)
    l_t = lax.broadcasted_iota(jnp.int32, (_SUB, _TAIL), 1)
    cnt_t = (row0 + s_t) * np.int32(_V) + l_t + np.int32(_NCH * _W)
    d_t = _race_d(xs_t, cnt_t, ntinv2)
    m2 = jnp.max(d_t, axis=1, keepdims=True)
    sel2 = jnp.min(
        jnp.where(d_t == m2, cnt_t, _BIG_IDX), axis=1, keepdims=True
    )

    sel = jnp.where(
        m2 > m1, sel2, jnp.where(m1 > m2, sel1, jnp.minimum(sel1, sel2))
    )
    col = sel - (row0 + lax.broadcasted_iota(jnp.int32, (_SUB, 1), 0)) * np.int32(_V)
    o_ref[...] = jnp.broadcast_to(col, (_SUB, 128))


def kernel(logits, temperatures):
    tinv = (np.float32(-np.log2(np.e)) / temperatures).reshape(_B, 1)
    out = pl.pallas_call(
        _rowgroup_body,
        grid=(_B // _SUB,),
        in_specs=[
            pl.BlockSpec((_SUB, _V), lambda g: (g, 0)),
            pl.BlockSpec((_SUB, 1), lambda g: (g, 0)),
        ],
        out_specs=pl.BlockSpec((_SUB, 128), lambda g: (g, 0)),
        out_shape=jax.ShapeDtypeStruct((_B, 128), jnp.int32),
    )(logits, tinv)
    return out[:, 0]
